# baseline (speedup 1.0000x reference)
import jax
import jax.numpy as jnp
from jax import lax
from jax.experimental import pallas as pl
from jax.experimental.pallas import tpu as pltpu
from jax.experimental.pallas import tpu_sc as plsc

N_NODES = 100000
N_EDGES = 3200000
MAX_DEG = 63

NUM_TILES = 16
LANES = 16

NPAD = 102400
SLICE = NPAD // NUM_TILES
HALF = SLICE // 2
ICHUNK = 1280

EDGES_PER_TILE = N_EDGES // NUM_TILES
EC = 4000
N_CHUNKS = EDGES_PER_TILE // EC


def _zero_acc(acc):
    zv = jnp.zeros((LANES,), jnp.float32)

    def body(i, _):
        base = i * (LANES * 8)
        for u in range(8):
            acc[pl.ds(base + u * LANES, LANES)] = zv
        return 0

    lax.fori_loop(0, NPAD // (LANES * 8), body, 0, unroll=False)


def _start_load(c, ebase, eidx_hbm, attr_hbm, ibuf, abuf, sem):
    start = pl.multiple_of(ebase + c * EC, 8)
    pltpu.async_copy(eidx_hbm.at[pl.ds(start, EC)], ibuf, sem)
    pltpu.async_copy(attr_hbm.at[pl.ds(start, EC)], abuf, sem)


def _wait_load(c, ebase, eidx_hbm, attr_hbm, ibuf, abuf, sem):
    start = pl.multiple_of(ebase + c * EC, 8)
    pltpu.make_async_copy(eidx_hbm.at[pl.ds(start, EC)], ibuf, sem).wait()
    pltpu.make_async_copy(attr_hbm.at[pl.ds(start, EC)], abuf, sem).wait()


def _scatter_chunk(ibuf, abuf, acc):
    UNR = 10

    def g(j, _):
        base = j * UNR * LANES
        ivs = [ibuf[pl.ds(base + u * LANES, LANES)] for u in range(UNR)]
        avs = [abuf[pl.ds(base + u * LANES, LANES)] for u in range(UNR)]
        for u in range(UNR):
            plsc.addupdate_scatter(acc, [ivs[u]], avs[u])
        return 0

    lax.fori_loop(0, EC // (LANES * UNR), g, 0, unroll=False)


def _accumulate(sid, eidx_hbm, attr_hbm, ibuf0, abuf0, ibuf1, abuf1,
                acc, sem0, sem1):
    ebase = sid * EDGES_PER_TILE
    _start_load(0, ebase, eidx_hbm, attr_hbm, ibuf0, abuf0, sem0)

    def pair(k, _):
        c0 = k * 2
        _start_load(c0 + 1, ebase, eidx_hbm, attr_hbm, ibuf1, abuf1, sem1)
        _wait_load(c0, ebase, eidx_hbm, attr_hbm, ibuf0, abuf0, sem0)
        _scatter_chunk(ibuf0, abuf0, acc)

        @pl.when(c0 + 2 < N_CHUNKS)
        def _():
            _start_load(c0 + 2, ebase, eidx_hbm, attr_hbm, ibuf0, abuf0, sem0)

        _wait_load(c0 + 1, ebase, eidx_hbm, attr_hbm, ibuf1, abuf1, sem1)
        _scatter_chunk(ibuf1, abuf1, acc)
        return 0

    lax.fori_loop(0, N_CHUNKS // 2, pair, 0, unroll=False)


def _reduce(sid, acc, staging, tmp):
    own = sid * SLICE
    for r in range(1, NUM_TILES):
        o = lax.rem(sid + r, NUM_TILES)
        for q in range(2):
            src = pl.multiple_of(o * SLICE + q * HALF, 8)
            pltpu.sync_copy(acc.at[pl.ds(src, HALF)], staging.at[o])
            plsc.subcore_barrier()
            pltpu.sync_copy(staging.at[sid], tmp)

            def addb(j, _):
                s0 = j * 8 * LANES
                d0 = own + q * HALF + s0
                axs = [acc[pl.ds(d0 + u * LANES, LANES)] for u in range(8)]
                txs = [tmp[pl.ds(s0 + u * LANES, LANES)] for u in range(8)]
                for u in range(8):
                    acc[pl.ds(d0 + u * LANES, LANES)] = axs[u] + txs[u]
                return 0

            lax.fori_loop(0, HALF // (LANES * 8), addb, 0, unroll=False)
            plsc.subcore_barrier()


def _interp_write(sid, acc, lo_i, hi_i, lo_w, hi_w, oidx_hbm, ow_hbm):
    own = sid * SLICE
    for k in range(SLICE // ICHUNK):
        nbase = own + k * ICHUNK

        def interp(j, _):
            for u in range(2):
                s = (j * 2 + u) * LANES
                deg = acc[pl.ds(nbase + s, LANES)]
                deg = jnp.minimum(deg, jnp.float32(MAX_DEG))
                low = deg.astype(jnp.int32)
                frac = deg - low.astype(jnp.float32)
                hasf = frac > 0.0
                high = low + jnp.where(hasf, 1, 0)
                w_low = jnp.where(hasf, 1.0 - frac, jnp.float32(1.0))
                w_low = jnp.where(low == 0, jnp.float32(0.0), w_low)
                lo_i[pl.ds(s, LANES)] = low
                hi_i[pl.ds(s, LANES)] = high
                lo_w[pl.ds(s, LANES)] = w_low
                hi_w[pl.ds(s, LANES)] = frac
            return 0

        lax.fori_loop(0, ICHUNK // (LANES * 2), interp, 0, unroll=False)
        pltpu.sync_copy(lo_i, oidx_hbm.at[pl.ds(nbase, ICHUNK)])
        pltpu.sync_copy(hi_i, oidx_hbm.at[pl.ds(NPAD + nbase, ICHUNK)])
        pltpu.sync_copy(lo_w, ow_hbm.at[pl.ds(nbase, ICHUNK)])
        pltpu.sync_copy(hi_w, ow_hbm.at[pl.ds(NPAD + nbase, ICHUNK)])


def _degree_pipeline(sid, eidx_hbm, attr_hbm, oidx_hbm, ow_hbm,
                     ibuf0, abuf0, ibuf1, abuf1, tmp,
                     lo_i, hi_i, lo_w, hi_w, acc, staging, sem0, sem1):
    with jax.named_scope("ph_zero"):
        _zero_acc(acc)
    with jax.named_scope("ph_accum"):
        _accumulate(sid, eidx_hbm, attr_hbm, ibuf0, abuf0, ibuf1, abuf1,
                    acc, sem0, sem1)
        plsc.subcore_barrier()
    with jax.named_scope("ph_reduce"):
        _reduce(sid, acc, staging, tmp)
    with jax.named_scope("ph_interp"):
        _interp_write(sid, acc, lo_i, hi_i, lo_w, hi_w, oidx_hbm, ow_hbm)


def _sc_body(attr_hbm, dst_hbm, src_hbm,
             in_idx_hbm, in_w_hbm, out_idx_hbm, out_w_hbm,
             ibuf0, abuf0, ibuf1, abuf1, tmp,
             lo_i, hi_i, lo_w, hi_w, acc, staging, sem0, sem1):
    cid = lax.axis_index("c")
    sid = lax.axis_index("s")

    @pl.when(cid == 0)
    def _():
        _degree_pipeline(sid, dst_hbm, attr_hbm, in_idx_hbm, in_w_hbm,
                         ibuf0, abuf0, ibuf1, abuf1, tmp,
                         lo_i, hi_i, lo_w, hi_w, acc, staging, sem0, sem1)

    @pl.when(cid == 1)
    def _():
        _degree_pipeline(sid, src_hbm, attr_hbm, out_idx_hbm, out_w_hbm,
                         ibuf0, abuf0, ibuf1, abuf1, tmp,
                         lo_i, hi_i, lo_w, hi_w, acc, staging, sem0, sem1)


@jax.jit
def kernel(edge_attr, edge_index):
    dst = edge_index[1]
    src = edge_index[0]

    mesh = plsc.VectorSubcoreMesh(core_axis_name="c", subcore_axis_name="s")
    run = pl.kernel(
        _sc_body,
        out_type=[
            jax.ShapeDtypeStruct((NPAD * 2,), jnp.int32),
            jax.ShapeDtypeStruct((NPAD * 2,), jnp.float32),
            jax.ShapeDtypeStruct((NPAD * 2,), jnp.int32),
            jax.ShapeDtypeStruct((NPAD * 2,), jnp.float32),
        ],
        mesh=mesh,
        compiler_params=pltpu.CompilerParams(needs_layout_passes=False),
        scratch_types=[
            pltpu.VMEM((EC,), jnp.int32),
            pltpu.VMEM((EC,), jnp.float32),
            pltpu.VMEM((EC,), jnp.int32),
            pltpu.VMEM((EC,), jnp.float32),
            pltpu.VMEM((HALF,), jnp.float32),
            pltpu.VMEM((ICHUNK,), jnp.int32),
            pltpu.VMEM((ICHUNK,), jnp.int32),
            pltpu.VMEM((ICHUNK,), jnp.float32),
            pltpu.VMEM((ICHUNK,), jnp.float32),
            pltpu.VMEM((NPAD,), jnp.float32),
            pltpu.VMEM_SHARED((NUM_TILES, HALF), jnp.float32),
            pltpu.SemaphoreType.DMA,
            pltpu.SemaphoreType.DMA,
        ],
    )
    in_idx, in_w, out_idx, out_w = run(edge_attr, dst, src)

    def planes_to_pairs(flat):
        return jnp.concatenate(
            [flat[:N_NODES, None], flat[NPAD:NPAD + N_NODES, None]], axis=1)

    return (planes_to_pairs(in_idx), planes_to_pairs(in_w),
            planes_to_pairs(out_idx), planes_to_pairs(out_w))


# speedup vs baseline: 1.0814x; 1.0814x over previous
import jax
import jax.numpy as jnp
from jax import lax
from jax.experimental import pallas as pl
from jax.experimental.pallas import tpu as pltpu
from jax.experimental.pallas import tpu_sc as plsc

N_NODES = 100000
N_EDGES = 3200000
MAX_DEG = 63

NUM_TILES = 16
LANES = 16

NPAD = 102400
SLICE = NPAD // NUM_TILES
HALF = SLICE // 2
ICHUNK = 1280

EDGES_PER_TILE = N_EDGES // NUM_TILES
EC = 4000
N_CHUNKS = EDGES_PER_TILE // EC


def _zero_acc(acc):
    zv = jnp.zeros((LANES,), jnp.float32)

    def body(i, _):
        base = i * (LANES * 16)
        for u in range(16):
            acc[pl.ds(base + u * LANES, LANES)] = zv
        return 0

    lax.fori_loop(0, NPAD // (LANES * 16), body, 0, unroll=False)


def _start_load(c, ebase, eidx_hbm, attr_hbm, ibuf, abuf, sem):
    start = pl.multiple_of(ebase + c * EC, 8)
    pltpu.async_copy(eidx_hbm.at[pl.ds(start, EC)], ibuf, sem)
    pltpu.async_copy(attr_hbm.at[pl.ds(start, EC)], abuf, sem)


def _wait_load(c, ebase, eidx_hbm, attr_hbm, ibuf, abuf, sem):
    start = pl.multiple_of(ebase + c * EC, 8)
    pltpu.make_async_copy(eidx_hbm.at[pl.ds(start, EC)], ibuf, sem).wait()
    pltpu.make_async_copy(attr_hbm.at[pl.ds(start, EC)], abuf, sem).wait()


UNR = 10


def _load_block(ibuf, abuf, j):
    base = j * UNR * LANES
    ivs = [ibuf[pl.ds(base + u * LANES, LANES)] for u in range(UNR)]
    avs = [abuf[pl.ds(base + u * LANES, LANES)] for u in range(UNR)]
    return ivs + avs


def _scatter_chunk(ibuf, abuf, acc):
    nblk = EC // (LANES * UNR)
    carry0 = _load_block(ibuf, abuf, 0)

    def g(j, carry):
        nxt = _load_block(ibuf, abuf, j + 1)
        for u in range(UNR):
            plsc.addupdate_scatter(acc, [carry[u]], carry[UNR + u])
        return nxt

    carry = lax.fori_loop(0, nblk - 1, g, carry0, unroll=False)
    for u in range(UNR):
        plsc.addupdate_scatter(acc, [carry[u]], carry[UNR + u])


def _accumulate(sid, eidx_hbm, attr_hbm, ibuf0, abuf0, ibuf1, abuf1,
                acc, sem0, sem1):
    ebase = sid * EDGES_PER_TILE
    _start_load(0, ebase, eidx_hbm, attr_hbm, ibuf0, abuf0, sem0)

    def pair(k, _):
        c0 = k * 2
        _start_load(c0 + 1, ebase, eidx_hbm, attr_hbm, ibuf1, abuf1, sem1)
        _wait_load(c0, ebase, eidx_hbm, attr_hbm, ibuf0, abuf0, sem0)
        _scatter_chunk(ibuf0, abuf0, acc)

        @pl.when(c0 + 2 < N_CHUNKS)
        def _():
            _start_load(c0 + 2, ebase, eidx_hbm, attr_hbm, ibuf0, abuf0, sem0)

        _wait_load(c0 + 1, ebase, eidx_hbm, attr_hbm, ibuf1, abuf1, sem1)
        _scatter_chunk(ibuf1, abuf1, acc)
        return 0

    lax.fori_loop(0, N_CHUNKS // 2, pair, 0, unroll=False)


def _reduce(sid, acc, staging, tmp):
    own = sid * SLICE
    for r in range(1, NUM_TILES):
        o = lax.rem(sid + r, NUM_TILES)
        for q in range(2):
            src = pl.multiple_of(o * SLICE + q * HALF, 8)
            pltpu.sync_copy(acc.at[pl.ds(src, HALF)], staging.at[o])
            plsc.subcore_barrier()
            pltpu.sync_copy(staging.at[sid], tmp)

            def addb(j, _):
                s0 = j * 8 * LANES
                d0 = own + q * HALF + s0
                txs = [tmp[pl.ds(s0 + u * LANES, LANES)] for u in range(8)]
                for u in range(8):
                    plsc.addupdate(acc.at[pl.ds(d0 + u * LANES, LANES)], txs[u])
                return 0

            lax.fori_loop(0, HALF // (LANES * 8), addb, 0, unroll=False)
            plsc.subcore_barrier()


def _interp_write(sid, acc, lo_i, hi_i, lo_w, hi_w, oidx_hbm, ow_hbm):
    own = sid * SLICE
    for k in range(SLICE // ICHUNK):
        nbase = own + k * ICHUNK

        def interp(j, _):
            s0 = j * 4 * LANES
            degs = [acc[pl.ds(nbase + s0 + u * LANES, LANES)] for u in range(4)]
            for u in range(4):
                s = s0 + u * LANES
                deg = jnp.minimum(degs[u], jnp.float32(MAX_DEG))
                low = deg.astype(jnp.int32)
                frac = deg - low.astype(jnp.float32)
                hasf = frac > 0.0
                high = low + jnp.where(hasf, 1, 0)
                w_low = jnp.where(hasf, 1.0 - frac, jnp.float32(1.0))
                w_low = jnp.where(low == 0, jnp.float32(0.0), w_low)
                lo_i[pl.ds(s, LANES)] = low
                hi_i[pl.ds(s, LANES)] = high
                lo_w[pl.ds(s, LANES)] = w_low
                hi_w[pl.ds(s, LANES)] = frac
            return 0

        lax.fori_loop(0, ICHUNK // (LANES * 4), interp, 0, unroll=False)
        pltpu.sync_copy(lo_i, oidx_hbm.at[pl.ds(nbase, ICHUNK)])
        pltpu.sync_copy(hi_i, oidx_hbm.at[pl.ds(NPAD + nbase, ICHUNK)])
        pltpu.sync_copy(lo_w, ow_hbm.at[pl.ds(nbase, ICHUNK)])
        pltpu.sync_copy(hi_w, ow_hbm.at[pl.ds(NPAD + nbase, ICHUNK)])


def _degree_pipeline(sid, eidx_hbm, attr_hbm, oidx_hbm, ow_hbm,
                     ibuf0, abuf0, ibuf1, abuf1, tmp,
                     lo_i, hi_i, lo_w, hi_w, acc, staging, sem0, sem1):
    with jax.named_scope("ph_zero"):
        _zero_acc(acc)
    with jax.named_scope("ph_accum"):
        _accumulate(sid, eidx_hbm, attr_hbm, ibuf0, abuf0, ibuf1, abuf1,
                    acc, sem0, sem1)
        plsc.subcore_barrier()
    with jax.named_scope("ph_reduce"):
        _reduce(sid, acc, staging, tmp)
    with jax.named_scope("ph_interp"):
        _interp_write(sid, acc, lo_i, hi_i, lo_w, hi_w, oidx_hbm, ow_hbm)


def _sc_body(attr_hbm, dst_hbm, src_hbm,
             in_idx_hbm, in_w_hbm, out_idx_hbm, out_w_hbm,
             ibuf0, abuf0, ibuf1, abuf1, tmp,
             lo_i, hi_i, lo_w, hi_w, acc, staging, sem0, sem1):
    cid = lax.axis_index("c")
    sid = lax.axis_index("s")

    @pl.when(cid == 0)
    def _():
        _degree_pipeline(sid, dst_hbm, attr_hbm, in_idx_hbm, in_w_hbm,
                         ibuf0, abuf0, ibuf1, abuf1, tmp,
                         lo_i, hi_i, lo_w, hi_w, acc, staging, sem0, sem1)

    @pl.when(cid == 1)
    def _():
        _degree_pipeline(sid, src_hbm, attr_hbm, out_idx_hbm, out_w_hbm,
                         ibuf0, abuf0, ibuf1, abuf1, tmp,
                         lo_i, hi_i, lo_w, hi_w, acc, staging, sem0, sem1)


@jax.jit
def kernel(edge_attr, edge_index):
    dst = edge_index[1]
    src = edge_index[0]

    mesh = plsc.VectorSubcoreMesh(core_axis_name="c", subcore_axis_name="s")
    run = pl.kernel(
        _sc_body,
        out_type=[
            jax.ShapeDtypeStruct((NPAD * 2,), jnp.int32),
            jax.ShapeDtypeStruct((NPAD * 2,), jnp.float32),
            jax.ShapeDtypeStruct((NPAD * 2,), jnp.int32),
            jax.ShapeDtypeStruct((NPAD * 2,), jnp.float32),
        ],
        mesh=mesh,
        compiler_params=pltpu.CompilerParams(needs_layout_passes=False),
        scratch_types=[
            pltpu.VMEM((EC,), jnp.int32),
            pltpu.VMEM((EC,), jnp.float32),
            pltpu.VMEM((EC,), jnp.int32),
            pltpu.VMEM((EC,), jnp.float32),
            pltpu.VMEM((HALF,), jnp.float32),
            pltpu.VMEM((ICHUNK,), jnp.int32),
            pltpu.VMEM((ICHUNK,), jnp.int32),
            pltpu.VMEM((ICHUNK,), jnp.float32),
            pltpu.VMEM((ICHUNK,), jnp.float32),
            pltpu.VMEM((NPAD,), jnp.float32),
            pltpu.VMEM_SHARED((NUM_TILES, HALF), jnp.float32),
            pltpu.SemaphoreType.DMA,
            pltpu.SemaphoreType.DMA,
        ],
    )
    in_idx, in_w, out_idx, out_w = run(edge_attr, dst, src)

    def planes_to_pairs(flat):
        return jnp.concatenate(
            [flat[:N_NODES, None], flat[NPAD:NPAD + N_NODES, None]], axis=1)

    return (planes_to_pairs(in_idx), planes_to_pairs(in_w),
            planes_to_pairs(out_idx), planes_to_pairs(out_w))


# speedup vs baseline: 1.1948x; 1.1048x over previous
import jax
import jax.numpy as jnp
from jax import lax
from jax.experimental import pallas as pl
from jax.experimental.pallas import tpu as pltpu
from jax.experimental.pallas import tpu_sc as plsc

N_NODES = 100000
N_EDGES = 3200000
MAX_DEG = 63

NUM_TILES = 16
LANES = 16
ROW = 128
NROWS = N_EDGES // ROW

NPAD = 102400
SLICE = NPAD // NUM_TILES
HALF = SLICE // 2
ICHUNK = 1280

ROWS_PER_TILE = 1560
REM_ROWS = NROWS - NUM_TILES * ROWS_PER_TILE
CROWS = 24
N_CHUNKS = ROWS_PER_TILE // CROWS
GPR = ROW // LANES


def _zero_acc(acc):
    zv = jnp.zeros((LANES,), jnp.float32)

    def body(i, _):
        base = i * (LANES * 16)
        for u in range(16):
            acc[pl.ds(base + u * LANES, LANES)] = zv
        return 0

    lax.fori_loop(0, NPAD // (LANES * 16), body, 0, unroll=False)


def _start_load(r0, nrows, ridx, eidx_hbm, attr_hbm, ibuf, abuf, sem):
    pltpu.async_copy(eidx_hbm.at[pl.ds(r0, nrows), ridx], ibuf, sem)
    pltpu.async_copy(attr_hbm.at[pl.ds(r0, nrows)], abuf, sem)


def _wait_load(r0, nrows, ridx, eidx_hbm, attr_hbm, ibuf, abuf, sem):
    pltpu.make_async_copy(eidx_hbm.at[pl.ds(r0, nrows), ridx], ibuf, sem).wait()
    pltpu.make_async_copy(attr_hbm.at[pl.ds(r0, nrows)], abuf, sem).wait()


def _load_row(ibuf, abuf, r):
    ivs = [ibuf[r, pl.ds(u * LANES, LANES)] for u in range(GPR)]
    avs = [abuf[r, pl.ds(u * LANES, LANES)] for u in range(GPR)]
    return ivs + avs


def _scatter_rows(nrows, ibuf, abuf, acc):
    carry0 = _load_row(ibuf, abuf, 0)

    def g(r, carry):
        nxt = _load_row(ibuf, abuf, r + 1)
        for u in range(GPR):
            plsc.addupdate_scatter(acc, [carry[u]], carry[GPR + u])
        return nxt

    carry = lax.fori_loop(0, nrows - 1, g, carry0, unroll=False)
    for u in range(GPR):
        plsc.addupdate_scatter(acc, [carry[u]], carry[GPR + u])


def _accumulate(sid, ridx, eidx_hbm, attr_hbm, ibuf0, abuf0, ibuf1, abuf1,
                acc, sem0, sem1):
    row_base = sid * ROWS_PER_TILE

    def start(c, ibuf, abuf, sem):
        _start_load(row_base + c * CROWS, CROWS, ridx,
                    eidx_hbm, attr_hbm, ibuf, abuf, sem)

    def wait(c, ibuf, abuf, sem):
        _wait_load(row_base + c * CROWS, CROWS, ridx,
                   eidx_hbm, attr_hbm, ibuf, abuf, sem)

    start(0, ibuf0, abuf0, sem0)

    def pair(k, _):
        c0 = k * 2
        start(c0 + 1, ibuf1, abuf1, sem1)
        wait(c0, ibuf0, abuf0, sem0)
        _scatter_rows(CROWS, ibuf0, abuf0, acc)

        @pl.when(c0 + 2 < N_CHUNKS)
        def _():
            start(c0 + 2, ibuf0, abuf0, sem0)

        wait(c0 + 1, ibuf1, abuf1, sem1)
        _scatter_rows(CROWS, ibuf1, abuf1, acc)
        return 0

    lax.fori_loop(0, N_CHUNKS // 2, pair, 0, unroll=False)
    wait(N_CHUNKS - 1, ibuf0, abuf0, sem0)
    _scatter_rows(CROWS, ibuf0, abuf0, acc)

    @pl.when(sid == 0)
    def _():
        base = NUM_TILES * ROWS_PER_TILE
        for r0, n in ((base, CROWS), (base + CROWS, REM_ROWS - CROWS)):
            _start_load(r0, n, ridx, eidx_hbm, attr_hbm,
                        ibuf1.at[pl.ds(0, n)], abuf1.at[pl.ds(0, n)], sem1)
            _wait_load(r0, n, ridx, eidx_hbm, attr_hbm,
                       ibuf1.at[pl.ds(0, n)], abuf1.at[pl.ds(0, n)], sem1)
            _scatter_rows(n, ibuf1, abuf1, acc)


def _reduce(sid, acc, staging, tmp):
    own = sid * SLICE
    for r in range(1, NUM_TILES):
        o = lax.rem(sid + r, NUM_TILES)
        for q in range(2):
            src = pl.multiple_of(o * SLICE + q * HALF, 8)
            pltpu.sync_copy(acc.at[pl.ds(src, HALF)], staging.at[o])
            plsc.subcore_barrier()
            pltpu.sync_copy(staging.at[sid], tmp)

            def addb(j, _):
                s0 = j * 8 * LANES
                d0 = own + q * HALF + s0
                txs = [tmp[pl.ds(s0 + u * LANES, LANES)] for u in range(8)]
                for u in range(8):
                    plsc.addupdate(acc.at[pl.ds(d0 + u * LANES, LANES)], txs[u])
                return 0

            lax.fori_loop(0, HALF // (LANES * 8), addb, 0, unroll=False)
            plsc.subcore_barrier()


def _interp_write(sid, acc, lo_i, hi_i, lo_w, hi_w, oidx_hbm, ow_hbm):
    own = sid * SLICE
    for k in range(SLICE // ICHUNK):
        nbase = own + k * ICHUNK

        def interp(j, _):
            s0 = j * 4 * LANES
            degs = [acc[pl.ds(nbase + s0 + u * LANES, LANES)] for u in range(4)]
            for u in range(4):
                s = s0 + u * LANES
                deg = jnp.minimum(degs[u], jnp.float32(MAX_DEG))
                low = deg.astype(jnp.int32)
                frac = deg - low.astype(jnp.float32)
                hasf = frac > 0.0
                high = low + jnp.where(hasf, 1, 0)
                w_low = jnp.where(hasf, 1.0 - frac, jnp.float32(1.0))
                w_low = jnp.where(low == 0, jnp.float32(0.0), w_low)
                lo_i[pl.ds(s, LANES)] = low
                hi_i[pl.ds(s, LANES)] = high
                lo_w[pl.ds(s, LANES)] = w_low
                hi_w[pl.ds(s, LANES)] = frac
            return 0

        lax.fori_loop(0, ICHUNK // (LANES * 4), interp, 0, unroll=False)
        pltpu.sync_copy(lo_i, oidx_hbm.at[pl.ds(nbase, ICHUNK)])
        pltpu.sync_copy(hi_i, oidx_hbm.at[pl.ds(NPAD + nbase, ICHUNK)])
        pltpu.sync_copy(lo_w, ow_hbm.at[pl.ds(nbase, ICHUNK)])
        pltpu.sync_copy(hi_w, ow_hbm.at[pl.ds(NPAD + nbase, ICHUNK)])


def _degree_pipeline(sid, ridx, eidx_hbm, attr_hbm, oidx_hbm, ow_hbm,
                     ibuf0, abuf0, ibuf1, abuf1, tmp,
                     lo_i, hi_i, lo_w, hi_w, acc, staging, sem0, sem1):
    with jax.named_scope("ph_zero"):
        _zero_acc(acc)
    with jax.named_scope("ph_accum"):
        _accumulate(sid, ridx, eidx_hbm, attr_hbm, ibuf0, abuf0, ibuf1, abuf1,
                    acc, sem0, sem1)
        plsc.subcore_barrier()
    with jax.named_scope("ph_reduce"):
        _reduce(sid, acc, staging, tmp)
    with jax.named_scope("ph_interp"):
        _interp_write(sid, acc, lo_i, hi_i, lo_w, hi_w, oidx_hbm, ow_hbm)


def _sc_body(eidx_hbm, attr_hbm,
             in_idx_hbm, in_w_hbm, out_idx_hbm, out_w_hbm,
             ibuf0, abuf0, ibuf1, abuf1, tmp,
             lo_i, hi_i, lo_w, hi_w, acc, staging, sem0, sem1):
    cid = lax.axis_index("c")
    sid = lax.axis_index("s")

    @pl.when(cid == 0)
    def _():
        _degree_pipeline(sid, 1, eidx_hbm, attr_hbm, in_idx_hbm, in_w_hbm,
                         ibuf0, abuf0, ibuf1, abuf1, tmp,
                         lo_i, hi_i, lo_w, hi_w, acc, staging, sem0, sem1)

    @pl.when(cid == 1)
    def _():
        _degree_pipeline(sid, 0, eidx_hbm, attr_hbm, out_idx_hbm, out_w_hbm,
                         ibuf0, abuf0, ibuf1, abuf1, tmp,
                         lo_i, hi_i, lo_w, hi_w, acc, staging, sem0, sem1)


@jax.jit
def kernel(edge_attr, edge_index):
    eall = edge_index.reshape(2, NROWS, ROW).transpose(1, 0, 2)
    attr = edge_attr.reshape(NROWS, ROW)

    mesh = plsc.VectorSubcoreMesh(core_axis_name="c", subcore_axis_name="s")
    run = pl.kernel(
        _sc_body,
        out_type=[
            jax.ShapeDtypeStruct((NPAD * 2,), jnp.int32),
            jax.ShapeDtypeStruct((NPAD * 2,), jnp.float32),
            jax.ShapeDtypeStruct((NPAD * 2,), jnp.int32),
            jax.ShapeDtypeStruct((NPAD * 2,), jnp.float32),
        ],
        mesh=mesh,
        compiler_params=pltpu.CompilerParams(needs_layout_passes=False),
        scratch_types=[
            pltpu.VMEM((CROWS, ROW), jnp.int32),
            pltpu.VMEM((CROWS, ROW), jnp.float32),
            pltpu.VMEM((CROWS, ROW), jnp.int32),
            pltpu.VMEM((CROWS, ROW), jnp.float32),
            pltpu.VMEM((HALF,), jnp.float32),
            pltpu.VMEM((ICHUNK,), jnp.int32),
            pltpu.VMEM((ICHUNK,), jnp.int32),
            pltpu.VMEM((ICHUNK,), jnp.float32),
            pltpu.VMEM((ICHUNK,), jnp.float32),
            pltpu.VMEM((NPAD,), jnp.float32),
            pltpu.VMEM_SHARED((NUM_TILES, HALF), jnp.float32),
            pltpu.SemaphoreType.DMA,
            pltpu.SemaphoreType.DMA,
        ],
    )
    in_idx, in_w, out_idx, out_w = run(eall, attr)

    def planes_to_pairs(flat):
        return jnp.concatenate(
            [flat[:N_NODES, None], flat[NPAD:NPAD + N_NODES, None]], axis=1)

    return (planes_to_pairs(in_idx), planes_to_pairs(in_w),
            planes_to_pairs(out_idx), planes_to_pairs(out_w))


# speedup vs baseline: 1.2221x; 1.0228x over previous
import jax
import jax.numpy as jnp
from jax import lax
from jax.experimental import pallas as pl
from jax.experimental.pallas import tpu as pltpu
from jax.experimental.pallas import tpu_sc as plsc

N_NODES = 100000
N_EDGES = 3200000
MAX_DEG = 63

NUM_TILES = 16
LANES = 16
ROW = 128
NROWS = N_EDGES // ROW

NPAD = 102400
SLICE = NPAD // NUM_TILES
HALF = SLICE // 2
ICHUNK = 1280

ROWS_PER_TILE = 1560
REM_ROWS = NROWS - NUM_TILES * ROWS_PER_TILE
CROWS = 24
N_CHUNKS = ROWS_PER_TILE // CROWS
GPR = ROW // LANES


def _zero_acc(acc):
    zv = jnp.zeros((LANES,), jnp.float32)

    def body(i, _):
        base = i * (LANES * 16)
        for u in range(16):
            acc[pl.ds(base + u * LANES, LANES)] = zv
        return 0

    lax.fori_loop(0, NPAD // (LANES * 16), body, 0, unroll=False)


def _start_load(r0, nrows, ridx, eidx_hbm, attr_hbm, ibuf, abuf, sem):
    pltpu.async_copy(eidx_hbm.at[pl.ds(r0, nrows), ridx], ibuf, sem)
    pltpu.async_copy(attr_hbm.at[pl.ds(r0, nrows)], abuf, sem)


def _wait_load(r0, nrows, ridx, eidx_hbm, attr_hbm, ibuf, abuf, sem):
    pltpu.make_async_copy(eidx_hbm.at[pl.ds(r0, nrows), ridx], ibuf, sem).wait()
    pltpu.make_async_copy(attr_hbm.at[pl.ds(r0, nrows)], abuf, sem).wait()


def _load_row(ibuf, abuf, r):
    ivs = [ibuf[r, pl.ds(u * LANES, LANES)] for u in range(GPR)]
    avs = [abuf[r, pl.ds(u * LANES, LANES)] for u in range(GPR)]
    return ivs + avs


def _scatter_rows(nrows, ibuf, abuf, acc):
    carry0 = _load_row(ibuf, abuf, 0)

    def g(r, carry):
        nxt = _load_row(ibuf, abuf, r + 1)
        for u in range(GPR):
            plsc.addupdate_scatter(acc, [carry[u]], carry[GPR + u])
        return nxt

    carry = lax.fori_loop(0, nrows - 1, g, carry0, unroll=False)
    for u in range(GPR):
        plsc.addupdate_scatter(acc, [carry[u]], carry[GPR + u])


def _accumulate(sid, ridx, eidx_hbm, attr_hbm, ibuf0, abuf0, ibuf1, abuf1,
                acc, sem0, sem1):
    row_base = sid * ROWS_PER_TILE

    def start(c, ibuf, abuf, sem):
        _start_load(row_base + c * CROWS, CROWS, ridx,
                    eidx_hbm, attr_hbm, ibuf, abuf, sem)

    def wait(c, ibuf, abuf, sem):
        _wait_load(row_base + c * CROWS, CROWS, ridx,
                   eidx_hbm, attr_hbm, ibuf, abuf, sem)

    start(0, ibuf0, abuf0, sem0)

    def pair(k, _):
        c0 = k * 2
        start(c0 + 1, ibuf1, abuf1, sem1)
        wait(c0, ibuf0, abuf0, sem0)
        _scatter_rows(CROWS, ibuf0, abuf0, acc)

        @pl.when(c0 + 2 < N_CHUNKS)
        def _():
            start(c0 + 2, ibuf0, abuf0, sem0)

        wait(c0 + 1, ibuf1, abuf1, sem1)
        _scatter_rows(CROWS, ibuf1, abuf1, acc)
        return 0

    lax.fori_loop(0, N_CHUNKS // 2, pair, 0, unroll=False)
    wait(N_CHUNKS - 1, ibuf0, abuf0, sem0)
    _scatter_rows(CROWS, ibuf0, abuf0, acc)

    @pl.when(sid < REM_ROWS // 8)
    def _():
        base = NUM_TILES * ROWS_PER_TILE
        r0 = pl.multiple_of(base + sid * 8, 8)
        _start_load(r0, 8, ridx, eidx_hbm, attr_hbm,
                    ibuf1.at[pl.ds(0, 8)], abuf1.at[pl.ds(0, 8)], sem1)
        _wait_load(r0, 8, ridx, eidx_hbm, attr_hbm,
                   ibuf1.at[pl.ds(0, 8)], abuf1.at[pl.ds(0, 8)], sem1)
        _scatter_rows(8, ibuf1, abuf1, acc)


def _reduce(sid, acc, staging, tmp):
    own = sid * SLICE
    for i in range(2 * (NUM_TILES - 1)):
        r = 1 + i // 2
        q = i % 2
        o = lax.rem(sid + r, NUM_TILES)
        stg = staging.at[i % 2]
        src = pl.multiple_of(o * SLICE + q * HALF, 8)
        pltpu.sync_copy(acc.at[pl.ds(src, HALF)], stg.at[o])
        plsc.subcore_barrier()
        pltpu.sync_copy(stg.at[sid], tmp)

        def addb(j, _):
            s0 = j * 8 * LANES
            d0 = own + q * HALF + s0
            txs = [tmp[pl.ds(s0 + u * LANES, LANES)] for u in range(8)]
            for u in range(8):
                plsc.addupdate(acc.at[pl.ds(d0 + u * LANES, LANES)], txs[u])
            return 0

        lax.fori_loop(0, HALF // (LANES * 8), addb, 0, unroll=False)


def _interp_write(sid, acc, lo_i, hi_i, lo_w, hi_w, oidx_hbm, ow_hbm):
    own = sid * SLICE
    for k in range(SLICE // ICHUNK):
        nbase = own + k * ICHUNK

        def interp(j, _):
            s0 = j * 4 * LANES
            degs = [acc[pl.ds(nbase + s0 + u * LANES, LANES)] for u in range(4)]
            for u in range(4):
                s = s0 + u * LANES
                deg = jnp.minimum(degs[u], jnp.float32(MAX_DEG))
                low = deg.astype(jnp.int32)
                frac = deg - low.astype(jnp.float32)
                hasf = frac > 0.0
                high = low + jnp.where(hasf, 1, 0)
                w_low = jnp.where(hasf, 1.0 - frac, jnp.float32(1.0))
                w_low = jnp.where(low == 0, jnp.float32(0.0), w_low)
                lo_i[pl.ds(s, LANES)] = low
                hi_i[pl.ds(s, LANES)] = high
                lo_w[pl.ds(s, LANES)] = w_low
                hi_w[pl.ds(s, LANES)] = frac
            return 0

        lax.fori_loop(0, ICHUNK // (LANES * 4), interp, 0, unroll=False)
        pltpu.sync_copy(lo_i, oidx_hbm.at[pl.ds(nbase, ICHUNK)])
        pltpu.sync_copy(hi_i, oidx_hbm.at[pl.ds(NPAD + nbase, ICHUNK)])
        pltpu.sync_copy(lo_w, ow_hbm.at[pl.ds(nbase, ICHUNK)])
        pltpu.sync_copy(hi_w, ow_hbm.at[pl.ds(NPAD + nbase, ICHUNK)])


def _degree_pipeline(sid, ridx, eidx_hbm, attr_hbm, oidx_hbm, ow_hbm,
                     ibuf0, abuf0, ibuf1, abuf1, tmp,
                     lo_i, hi_i, lo_w, hi_w, acc, staging, sem0, sem1):
    with jax.named_scope("ph_zero"):
        _zero_acc(acc)
    with jax.named_scope("ph_accum"):
        _accumulate(sid, ridx, eidx_hbm, attr_hbm, ibuf0, abuf0, ibuf1, abuf1,
                    acc, sem0, sem1)
        plsc.subcore_barrier()
    with jax.named_scope("ph_reduce"):
        _reduce(sid, acc, staging, tmp)
    with jax.named_scope("ph_interp"):
        _interp_write(sid, acc, lo_i, hi_i, lo_w, hi_w, oidx_hbm, ow_hbm)


def _sc_body(eidx_hbm, attr_hbm,
             in_idx_hbm, in_w_hbm, out_idx_hbm, out_w_hbm,
             ibuf0, abuf0, ibuf1, abuf1, tmp,
             lo_i, hi_i, lo_w, hi_w, acc, staging, sem0, sem1):
    cid = lax.axis_index("c")
    sid = lax.axis_index("s")

    @pl.when(cid == 0)
    def _():
        _degree_pipeline(sid, 1, eidx_hbm, attr_hbm, in_idx_hbm, in_w_hbm,
                         ibuf0, abuf0, ibuf1, abuf1, tmp,
                         lo_i, hi_i, lo_w, hi_w, acc, staging, sem0, sem1)

    @pl.when(cid == 1)
    def _():
        _degree_pipeline(sid, 0, eidx_hbm, attr_hbm, out_idx_hbm, out_w_hbm,
                         ibuf0, abuf0, ibuf1, abuf1, tmp,
                         lo_i, hi_i, lo_w, hi_w, acc, staging, sem0, sem1)


@jax.jit
def kernel(edge_attr, edge_index):
    eall = edge_index.reshape(2, NROWS, ROW).transpose(1, 0, 2)
    attr = edge_attr.reshape(NROWS, ROW)

    mesh = plsc.VectorSubcoreMesh(core_axis_name="c", subcore_axis_name="s")
    run = pl.kernel(
        _sc_body,
        out_type=[
            jax.ShapeDtypeStruct((NPAD * 2,), jnp.int32),
            jax.ShapeDtypeStruct((NPAD * 2,), jnp.float32),
            jax.ShapeDtypeStruct((NPAD * 2,), jnp.int32),
            jax.ShapeDtypeStruct((NPAD * 2,), jnp.float32),
        ],
        mesh=mesh,
        compiler_params=pltpu.CompilerParams(needs_layout_passes=False),
        scratch_types=[
            pltpu.VMEM((CROWS, ROW), jnp.int32),
            pltpu.VMEM((CROWS, ROW), jnp.float32),
            pltpu.VMEM((CROWS, ROW), jnp.int32),
            pltpu.VMEM((CROWS, ROW), jnp.float32),
            pltpu.VMEM((HALF,), jnp.float32),
            pltpu.VMEM((ICHUNK,), jnp.int32),
            pltpu.VMEM((ICHUNK,), jnp.int32),
            pltpu.VMEM((ICHUNK,), jnp.float32),
            pltpu.VMEM((ICHUNK,), jnp.float32),
            pltpu.VMEM((NPAD,), jnp.float32),
            pltpu.VMEM_SHARED((2, NUM_TILES, HALF), jnp.float32),
            pltpu.SemaphoreType.DMA,
            pltpu.SemaphoreType.DMA,
        ],
    )
    in_idx, in_w, out_idx, out_w = run(eall, attr)

    def planes_to_pairs(flat):
        return jnp.concatenate(
            [flat[:N_NODES, None], flat[NPAD:NPAD + N_NODES, None]], axis=1)

    return (planes_to_pairs(in_idx), planes_to_pairs(in_w),
            planes_to_pairs(out_idx), planes_to_pairs(out_w))
